# baseline (speedup 1.0000x reference)
import functools

import jax
import jax.numpy as jnp
from jax import lax
from jax.experimental import pallas as pl
from jax.experimental.pallas import tpu as pltpu
from jax.experimental.pallas import tpu_sc as plsc

N = 10000
D = 128
M5 = 5
NU = 2500
ND = 7500
NU_P = 2560
ND_P = 7680
BD = 960
BN = 2000
EPS = 1e-5
NW = 32
N_P = 10240
PADI = 9999


def _feat_body(h_ref, sl16_ref, wh_ref, wsp_ref, bemb_ref, w1_ref,
               feat_ref, fw1_ref):
    f = (jnp.dot(h_ref[...], wh_ref[...], preferred_element_type=jnp.float32)
         + jnp.dot(sl16_ref[...], wsp_ref[...],
                   preferred_element_type=jnp.float32)
         + bemb_ref[...])
    feat_ref[...] = f
    fw1_ref[...] = jnp.dot(f, w1_ref[...], preferred_element_type=jnp.float32)


def _features(h, sl16, W_emb, b_emb, W1):
    wh = W_emb[:D, :]
    wsp = jnp.zeros((16, D), jnp.float32).at[:3, :].set(W_emb[D:, :])
    grid = N // BN
    return pl.pallas_call(
        _feat_body,
        grid=(grid,),
        in_specs=[
            pl.BlockSpec((BN, D), lambda i: (i, 0)),
            pl.BlockSpec((BN, 16), lambda i: (i, 0)),
            pl.BlockSpec((D, D), lambda i: (0, 0)),
            pl.BlockSpec((16, D), lambda i: (0, 0)),
            pl.BlockSpec((1, D), lambda i: (0, 0)),
            pl.BlockSpec((D, D), lambda i: (0, 0)),
        ],
        out_specs=[
            pl.BlockSpec((BN, D), lambda i: (i, 0)),
            pl.BlockSpec((BN, D), lambda i: (i, 0)),
        ],
        out_shape=[
            jax.ShapeDtypeStruct((N, D), jnp.float32),
            jax.ShapeDtypeStruct((N, D), jnp.float32),
        ],
    )(h, sl16, wh, wsp, b_emb.reshape(1, D), W1)


def _knn_body(sdp_ref, fwd_ref, sut_ref, fwub_ref, b1_ref, w2_ref, b2_ref,
              idx_ref, c_ref):
    sd0 = sdp_ref[:, 0:1]
    sd1 = sdp_ref[:, 1:2]
    sd2c = sdp_ref[:, 2:3]
    su0 = sut_ref[0:1, :]
    su1 = sut_ref[1:2, :]
    su2c = sut_ref[2:3, :]
    sdq = sd0 * sd0 + sd1 * sd1 + sd2c * sd2c
    suq = su0 * su0 + su1 * su1 + su2c * su2c
    cross = jnp.dot(sdp_ref[:, 0:8], sut_ref[...],
                    preferred_element_type=jnp.float32)
    d2 = sdq + suq - 2.0 * cross

    row = pl.program_id(0) * BD + jax.lax.broadcasted_iota(jnp.int32, (BD, 1), 0)
    valid = row < ND
    lane = jax.lax.broadcasted_iota(jnp.int32, (BD, NU_P), 1)

    fwd = fwd_ref[...]
    b1 = b1_ref[...]
    w2 = w2_ref[...]
    b2 = b2_ref[...]

    idx_cols = []
    c_cols = []
    for _ in range(M5):
        idxv = jnp.argmin(d2, axis=1).astype(jnp.int32).reshape(BD, 1)
        sel = lane == idxv
        d2 = jnp.where(sel, 1e30, d2)
        maskf = jnp.where(sel, 1.0, 0.0)
        g = jnp.dot(maskf, fwub_ref[...], preferred_element_type=jnp.float32)
        t = jnp.maximum(fwd - g + b1, 0.0)
        att = jnp.maximum(jnp.dot(t, w2, preferred_element_type=jnp.float32)
                          + b2, 0.0)
        c = jnp.exp(-att)
        idx_cols.append(jnp.where(valid, idxv, NU_P - 1))
        c_cols.append(c)

    zi = jnp.full((BD, 1), NU_P - 1, jnp.int32)
    zf = jnp.zeros((BD, 1), jnp.float32)
    idx_ref[...] = jnp.concatenate(idx_cols + [zi, zi, zi], axis=1)
    c_ref[...] = jnp.concatenate(c_cols + [zf, zf, zf], axis=1)


def _knn_agg(sdp, fwd, sut, fwu, b1, W2, b2):
    grid = ND_P // BD
    return pl.pallas_call(
        _knn_body,
        grid=(grid,),
        in_specs=[
            pl.BlockSpec((BD, 16), lambda i: (i, 0)),
            pl.BlockSpec((BD, D), lambda i: (i, 0)),
            pl.BlockSpec((8, NU_P), lambda i: (0, 0)),
            pl.BlockSpec((NU_P, D), lambda i: (0, 0)),
            pl.BlockSpec((1, D), lambda i: (0, 0)),
            pl.BlockSpec((D, 1), lambda i: (0, 0)),
            pl.BlockSpec((1, 1), lambda i: (0, 0)),
        ],
        out_specs=[
            pl.BlockSpec((BD, 8), lambda i: (i, 0)),
            pl.BlockSpec((BD, 8), lambda i: (i, 0)),
        ],
        out_shape=[
            jax.ShapeDtypeStruct((ND_P, 8), jnp.int32),
            jax.ShapeDtypeStruct((ND_P, 8), jnp.float32),
        ],
    )(sdp, fwd, sut, fwu, b1.reshape(1, D), W2, b2.reshape(1, 1))


def _agg_body(f_ref, s_ref, mx_ref, mn_ref, dg_ref, wo_ref, bo_ref,
              x1_ref, st_ref):
    pid = pl.program_id(0)

    @pl.when(pid == 0)
    def _init():
        st_ref[...] = jnp.zeros_like(st_ref)

    f = f_ref[...]
    s = s_ref[...]
    mx = mx_ref[...]
    mn = mn_ref[...]
    dg = dg_ref[...]
    mean_agg = f * (s / jnp.maximum(dg, 1.0))
    max_agg = f * jnp.where(f >= 0.0, mx, mn)
    cat = jnp.concatenate([mean_agg, max_agg], axis=1)
    agg = jnp.dot(cat, wo_ref[...], preferred_element_type=jnp.float32) \
        + bo_ref[...]
    agg = jnp.where(dg > 0.0, agg, 0.0)
    x1 = f + agg
    x1_ref[...] = x1
    st_ref[0:1, :] += jnp.sum(x1, axis=0, keepdims=True)
    st_ref[1:2, :] += jnp.sum(x1 * x1, axis=0, keepdims=True)


def _agg_stage(feat, s_g, mx_g, mn_g, dg_g, WO, bO):
    grid = N // BN
    return pl.pallas_call(
        _agg_body,
        grid=(grid,),
        in_specs=[
            pl.BlockSpec((BN, D), lambda i: (i, 0)),
            pl.BlockSpec((BN, 1), lambda i: (i, 0)),
            pl.BlockSpec((BN, 1), lambda i: (i, 0)),
            pl.BlockSpec((BN, 1), lambda i: (i, 0)),
            pl.BlockSpec((BN, 1), lambda i: (i, 0)),
            pl.BlockSpec((2 * D, D), lambda i: (0, 0)),
            pl.BlockSpec((1, D), lambda i: (0, 0)),
        ],
        out_specs=[
            pl.BlockSpec((BN, D), lambda i: (i, 0)),
            pl.BlockSpec((8, D), lambda i: (0, 0)),
        ],
        out_shape=[
            jax.ShapeDtypeStruct((N, D), jnp.float32),
            jax.ShapeDtypeStruct((8, D), jnp.float32),
        ],
    )(feat, s_g, mx_g, mn_g, dg_g, WO, bO.reshape(1, D))


def _ffn_body(x1_ref, st_ref, g1_ref, be1_ref, wf1_ref, bf1_ref, wf2_ref,
              bf2_ref, x2_ref, st2_ref):
    pid = pl.program_id(0)

    @pl.when(pid == 0)
    def _init():
        st2_ref[...] = jnp.zeros_like(st2_ref)

    mu = st_ref[0:1, :] / N
    var = st_ref[1:2, :] / N - mu * mu
    hh = (x1_ref[...] - mu) / jnp.sqrt(var + EPS) * g1_ref[...] + be1_ref[...]
    t = jnp.maximum(jnp.dot(hh, wf1_ref[...],
                            preferred_element_type=jnp.float32)
                    + bf1_ref[...], 0.0)
    y = jnp.dot(t, wf2_ref[...], preferred_element_type=jnp.float32) \
        + bf2_ref[...]
    x2 = hh + y
    x2_ref[...] = x2
    st2_ref[0:1, :] += jnp.sum(x2, axis=0, keepdims=True)
    st2_ref[1:2, :] += jnp.sum(x2 * x2, axis=0, keepdims=True)


def _ffn_stage(x1, st1, g1, be1, Wf1, bf1, Wf2, bf2):
    grid = N // BN
    return pl.pallas_call(
        _ffn_body,
        grid=(grid,),
        in_specs=[
            pl.BlockSpec((BN, D), lambda i: (i, 0)),
            pl.BlockSpec((8, D), lambda i: (0, 0)),
            pl.BlockSpec((1, D), lambda i: (0, 0)),
            pl.BlockSpec((1, D), lambda i: (0, 0)),
            pl.BlockSpec((D, 2 * D), lambda i: (0, 0)),
            pl.BlockSpec((1, 2 * D), lambda i: (0, 0)),
            pl.BlockSpec((2 * D, D), lambda i: (0, 0)),
            pl.BlockSpec((1, D), lambda i: (0, 0)),
        ],
        out_specs=[
            pl.BlockSpec((BN, D), lambda i: (i, 0)),
            pl.BlockSpec((8, D), lambda i: (0, 0)),
        ],
        out_shape=[
            jax.ShapeDtypeStruct((N, D), jnp.float32),
            jax.ShapeDtypeStruct((8, D), jnp.float32),
        ],
    )(x1, st1, g1.reshape(1, D), be1.reshape(1, D), Wf1,
      bf1.reshape(1, 2 * D), Wf2, bf2.reshape(1, D))


def _bn2_body(x2_ref, st_ref, g2_ref, be2_ref, out_ref):
    mu = st_ref[0:1, :] / N
    var = st_ref[1:2, :] / N - mu * mu
    out_ref[...] = (x2_ref[...] - mu) / jnp.sqrt(var + EPS) * g2_ref[...] \
        + be2_ref[...]


def _bn2_stage(x2, st2, g2, be2):
    grid = N // BN
    return pl.pallas_call(
        _bn2_body,
        grid=(grid,),
        in_specs=[
            pl.BlockSpec((BN, D), lambda i: (i, 0)),
            pl.BlockSpec((8, D), lambda i: (0, 0)),
            pl.BlockSpec((1, D), lambda i: (0, 0)),
            pl.BlockSpec((1, D), lambda i: (0, 0)),
        ],
        out_specs=pl.BlockSpec((BN, D), lambda i: (i, 0)),
        out_shape=jax.ShapeDtypeStruct((N, D), jnp.float32),
    )(x2, st2, g2.reshape(1, D), be2.reshape(1, D))


def _thr_compute(bits_ref, out_ref):
    bits = bits_ref[...]
    ridx = jax.lax.broadcasted_iota(jnp.int32, (N_P, 1), 0)
    j128 = jax.lax.broadcasted_iota(jnp.int32, (1, 128), 1)

    def count_ge(tvec):
        ge = jnp.where(bits >= tvec, 1.0, 0.0)
        return jnp.sum(ge, axis=0, keepdims=True)

    lo = jnp.int32(0)
    for step in (1 << 23, 1 << 16, 1 << 9, 1 << 2, 1):
        tvec = lo + j128 * step
        cnt = count_ge(tvec)
        jstar = jnp.max(jnp.where(cnt >= NU, j128, 0))
        lo = lo + jstar * step
    t = lo
    n_gt = jnp.sum(jnp.where(bits >= t + 1, 1.0, 0.0)).astype(jnp.int32)
    need = NU - n_gt
    tie = bits == t

    lo2 = jnp.int32(0)
    for step in (128, 1):
        xvec = lo2 + j128 * step
        cnt = jnp.sum(jnp.where(tie & (ridx >= xvec), 1.0, 0.0),
                      axis=0, keepdims=True)
        jstar = jnp.max(jnp.where(cnt >= need, j128, 0))
        lo2 = lo2 + jstar * step

    out_ref[0:1, :] = jnp.full((1, 128), t, jnp.int32)
    out_ref[1:2, :] = jnp.full((1, 128), lo2, jnp.int32)


def _threshold(bits_col):
    return pl.pallas_call(
        _thr_compute,
        grid=(1,),
        in_specs=[pl.BlockSpec((N_P, 1), lambda i: (0, 0))],
        out_specs=pl.BlockSpec((8, 128), lambda i: (0, 0)),
        out_shape=jax.ShapeDtypeStruct((8, 128), jnp.int32),
    )(bits_col)


def _sc_wid():
    return lax.axis_index("s") * 2 + lax.axis_index("c")


def _compact(bits, thr):
    mesh = plsc.VectorSubcoreMesh(core_axis_name="c", subcore_axis_name="s")

    @functools.partial(
        pl.kernel, mesh=mesh,
        out_type=[
            jax.ShapeDtypeStruct((1, NU_P), jnp.int32),
            jax.ShapeDtypeStruct((1, ND_P), jnp.int32),
        ],
        compiler_params=pltpu.CompilerParams(needs_layout_passes=False),
        scratch_types=[
            pltpu.VMEM((N,), jnp.int32),
            pltpu.VMEM((16,), jnp.int32),
            pltpu.VMEM((16,), jnp.int32),
            pltpu.VMEM((1, NU_P), jnp.int32),
            pltpu.VMEM((1, ND_P), jnp.int32),
        ],
    )
    def k(bits_hbm, thr_hbm, up_hbm, dn_hbm, bits_v, t_v, x_v, up_v, dn_v):
        w = _sc_wid()

        @pl.when(w == 0)
        def _():
            pltpu.sync_copy(bits_hbm, bits_v)
            pltpu.sync_copy(thr_hbm.at[pl.ds(0, 16)], t_v)
            pltpu.sync_copy(thr_hbm.at[pl.ds(128, 16)], x_v)
            t = t_v[...]
            x = x_v[...]
            lanes = lax.iota(jnp.int32, 16)
            zrow = jnp.zeros((16,), jnp.int32)
            padv = jnp.full((16,), PADI, jnp.int32)
            ones = jnp.full((16,), 1, jnp.int32)

            def fill_up(j, _):
                plsc.store_scatter(up_v, [zrow, lanes + j * 16], padv)
                return 0

            def fill_dn(j, _):
                plsc.store_scatter(dn_v, [zrow, lanes + j * 16], padv)
                return 0

            lax.fori_loop(0, NU_P // 16, fill_up, 0)
            lax.fori_loop(0, ND_P // 16, fill_dn, 0)

            def body(i, offs):
                uo, do = offs
                b16 = bits_v[pl.ds(i * 16, 16)]
                idx16 = lanes + i * 16
                m_up = (b16 > t) | ((b16 == t) & (idx16 >= x))
                m_dn = ~m_up
                cu = plsc.cumsum(m_up.astype(jnp.int32))
                cd = plsc.cumsum(m_dn.astype(jnp.int32))
                plsc.store_scatter(up_v, [zrow, uo + cu - 1], idx16,
                                   mask=m_up)
                plsc.store_scatter(dn_v, [zrow, do + cd - 1], idx16,
                                   mask=m_dn)
                nu16 = jnp.sum(jnp.where(m_up, ones, 0))
                return (uo + nu16, do + (16 - nu16))

            lax.fori_loop(0, N // 16, body,
                          (jnp.int32(0), jnp.int32(0)))
            pltpu.sync_copy(up_v, up_hbm)
            pltpu.sync_copy(dn_v, dn_hbm)

    return k(bits, thr)


def _sc_gather3(fw1, up_idx, dn_idx):
    bu = NU_P // NW
    bd = ND_P // NW
    mesh = plsc.VectorSubcoreMesh(core_axis_name="c", subcore_axis_name="s")

    @functools.partial(
        pl.kernel, mesh=mesh,
        out_type=[
            jax.ShapeDtypeStruct((NU_P, D), jnp.float32),
            jax.ShapeDtypeStruct((ND_P, D), jnp.float32),
        ],
        scratch_types=[
            pltpu.VMEM((bu,), jnp.int32),
            pltpu.VMEM((bd,), jnp.int32),
            pltpu.VMEM((bu, D), jnp.float32),
            pltpu.VMEM((bd, D), jnp.float32),
            pltpu.SemaphoreType.DMA,
        ],
    )
    def k(fw1_hbm, up_hbm, dn_hbm, fwu_hbm, fwd_hbm,
          iu_v, id_v, ru_v, rd_v, sem):
        w = _sc_wid()
        pltpu.sync_copy(up_hbm.at[pl.ds(w * bu, bu)], iu_v)
        pltpu.sync_copy(dn_hbm.at[pl.ds(w * bd, bd)], id_v)
        cu = pltpu.async_copy(fw1_hbm.at[iu_v], ru_v, sem)
        cd = pltpu.async_copy(fw1_hbm.at[id_v], rd_v, sem)
        cu.wait()
        cd.wait()
        pltpu.sync_copy(ru_v, fwu_hbm.at[pl.ds(w * bu, bu)])
        pltpu.sync_copy(rd_v, fwd_hbm.at[pl.ds(w * bd, bd)])

    return k(fw1, up_idx, dn_idx)


def _take16(x, i):
    dn = lax.GatherDimensionNumbers(offset_dims=(), collapsed_slice_dims=(0,),
                                    start_index_map=(0,))
    return lax.gather(x, i[:, None], dn, slice_sizes=(1,),
                      mode=lax.GatherScatterMode.PROMISE_IN_BOUNDS)


def _sc_segagg(idx5, c5):
    ew = ND_P * 8 // NW
    nit = ew // 16
    mesh = plsc.VectorSubcoreMesh(core_axis_name="c", subcore_axis_name="s")

    @functools.partial(
        pl.kernel, mesh=mesh,
        out_type=jax.ShapeDtypeStruct((NW * 4, NU_P), jnp.float32),
        compiler_params=pltpu.CompilerParams(needs_layout_passes=False),
        scratch_types=[
            pltpu.VMEM((ew,), jnp.int32),
            pltpu.VMEM((ew,), jnp.float32),
            pltpu.VMEM((4, NU_P), jnp.float32),
        ],
    )
    def k(idx_hbm, c_hbm, out_hbm, idx_v, c_v, part_v):
        w = _sc_wid()
        pltpu.sync_copy(idx_hbm.at[pl.ds(w * ew, ew)], idx_v)
        pltpu.sync_copy(c_hbm.at[pl.ds(w * ew, ew)], c_v)
        lanes = lax.iota(jnp.int32, 16)
        zrow = jnp.zeros((16,), jnp.int32)
        onerow = jnp.full((16,), 1, jnp.int32)
        tworow = jnp.full((16,), 2, jnp.int32)
        threerow = jnp.full((16,), 3, jnp.int32)
        onesf = jnp.full((16,), 1.0, jnp.float32)
        zz = jnp.zeros((16,), jnp.float32)
        tt = jnp.full((16,), 2.0, jnp.float32)
        lmask = (lanes % 8) < M5

        for j in range(NU_P // 16):
            part_v[0, pl.ds(j * 16, 16)] = zz
            part_v[1, pl.ds(j * 16, 16)] = zz
            part_v[2, pl.ds(j * 16, 16)] = tt
            part_v[3, pl.ds(j * 16, 16)] = zz

        def shift_ok(sk, d):
            idn = jnp.maximum(lanes - d, 0)
            km = (_take16(sk, idn) == sk) \
                & (lanes >= d)
            return idn, km

        def body(i, _):
            ii = idx_v[pl.ds(i * 16, 16)]
            cc = c_v[pl.ds(i * 16, 16)]
            plsc.addupdate_scatter(part_v, [zrow, ii], cc, mask=lmask)
            plsc.addupdate_scatter(part_v, [threerow, ii], onesf, mask=lmask)
            key = jnp.where(lmask, ii, NU_P - 1)
            sk, smx = plsc.sort_key_val(key, jnp.where(lmask, cc, 0.0))
            _, smn = plsc.sort_key_val(key, jnp.where(lmask, cc, 2.0))
            for d in (1, 2, 4, 8):
                idn, km = shift_ok(sk, d)
                smx = jnp.maximum(
                    smx, jnp.where(km, _take16(smx, idn),
                                   0.0))
                smn = jnp.minimum(
                    smn, jnp.where(km, _take16(smn, idn),
                                   2.0))
            nxt = jnp.minimum(lanes + 1, 15)
            islast = (lanes == 15) | (
                _take16(sk, nxt) != sk)
            curx = plsc.load_gather(part_v, [onerow, sk], mask=islast)
            plsc.store_scatter(part_v, [onerow, sk],
                               jnp.maximum(curx, smx), mask=islast)
            curn = plsc.load_gather(part_v, [tworow, sk], mask=islast)
            plsc.store_scatter(part_v, [tworow, sk],
                               jnp.minimum(curn, smn), mask=islast)
            return 0

        lax.fori_loop(0, nit, body, 0)
        pltpu.sync_copy(part_v, out_hbm.at[pl.ds(w * 4, 4)])

    return k(idx5, c5)


def _comb_body(p_ref, acc_ref):
    s = jnp.zeros((1, NU_P), jnp.float32)
    mx = jnp.zeros((1, NU_P), jnp.float32)
    mn = jnp.full((1, NU_P), 2.0, jnp.float32)
    dg = jnp.zeros((1, NU_P), jnp.float32)
    for w in range(NW):
        s = s + p_ref[4 * w:4 * w + 1, :]
        mx = jnp.maximum(mx, p_ref[4 * w + 1:4 * w + 2, :])
        mn = jnp.minimum(mn, p_ref[4 * w + 2:4 * w + 3, :])
        dg = dg + p_ref[4 * w + 3:4 * w + 4, :]
    acc_ref[0:1, :] = s
    acc_ref[1:2, :] = mx
    acc_ref[2:3, :] = mn
    acc_ref[3:4, :] = dg


def _combine(parts):
    return pl.pallas_call(
        _comb_body,
        grid=(1,),
        in_specs=[pl.BlockSpec((NW * 4, NU_P), lambda i: (0, 0))],
        out_specs=pl.BlockSpec((4, NU_P), lambda i: (0, 0)),
        out_shape=jax.ShapeDtypeStruct((4, NU_P), jnp.float32),
    )(parts)


def _sc_scatter4(acc, up_idx):
    bcols = N_P // NW
    mesh = plsc.VectorSubcoreMesh(core_axis_name="c", subcore_axis_name="s")

    @functools.partial(
        pl.kernel, mesh=mesh,
        out_type=jax.ShapeDtypeStruct((4, NW, bcols), jnp.float32),
        compiler_params=pltpu.CompilerParams(needs_layout_passes=False),
        scratch_types=[
            pltpu.VMEM((NU_P,), jnp.int32),
            pltpu.VMEM((4, NU_P), jnp.float32),
            pltpu.VMEM((4, bcols), jnp.float32),
        ],
    )
    def k(acc_hbm, up_hbm, out_hbm, idx_v, acc_v, buf_v):
        w = _sc_wid()
        lo = w * bcols
        pltpu.sync_copy(up_hbm, idx_v)
        pltpu.sync_copy(acc_hbm, acc_v)
        zz = jnp.zeros((16,), jnp.float32)
        lanes = lax.iota(jnp.int32, 16)
        for kk in range(4):
            for j in range(bcols // 16):
                buf_v[kk, pl.ds(j * 16, 16)] = zz

        def body(i, _):
            idx = idx_v[pl.ds(i * 16, 16)]
            pos = lanes + i * 16
            rel = idx - lo
            msk = (rel >= 0) & (rel < bcols) & (pos < NU)
            for kk in range(4):
                val = acc_v[kk, pl.ds(i * 16, 16)]
                plsc.store_scatter(buf_v,
                                   [jnp.full((16,), kk, jnp.int32), rel],
                                   val, mask=msk)
            return 0

        lax.fori_loop(0, NU_P // 16, body, 0)
        pltpu.sync_copy(buf_v, out_hbm.at[:, w])

    return k(acc, up_idx)


def kernel(h, s_l, scores, W_emb, b_emb, W1, b1, W2, b2, WO, bO, g1, be1,
           Wf1, bf1, Wf2, bf2, g2, be2):
    bits = jax.lax.bitcast_convert_type(scores, jnp.int32)
    bits_col = jnp.pad(bits, (0, N_P - N),
                       constant_values=-2**30).reshape(N_P, 1)
    sl16 = jnp.zeros((N, 16), jnp.float32).at[:, :3].set(s_l)

    thr = _threshold(bits_col).reshape(1024)
    nodes_up_p, nodes_down_p = _compact(bits, thr)

    feat, fw1 = _features(h, sl16, W_emb, b_emb, W1)
    nodes_up_p = nodes_up_p.reshape(NU_P)
    nodes_down_p = nodes_down_p.reshape(ND_P)
    nodes_up = nodes_up_p[:NU]

    sut = jnp.zeros((8, NU_P), jnp.float32)
    sut = sut.at[:3, :].set(1e6)
    sut = sut.at[:3, :NU].set(jnp.take(s_l, nodes_up, axis=0).T)
    fwu, fwd = _sc_gather3(fw1, nodes_up_p, nodes_down_p)
    sdp = jnp.take(sl16, nodes_down_p, axis=0)

    idx5, c5 = _knn_agg(sdp, fwd, sut, fwu, b1, W2, b2)
    parts = _sc_segagg(idx5.reshape(ND_P * 8), c5.reshape(ND_P * 8))
    acc = _combine(parts)

    sg4 = _sc_scatter4(acc, nodes_up_p).reshape(4, N_P)
    s_g = sg4[0, :N].reshape(N, 1)
    mx_g = sg4[1, :N].reshape(N, 1)
    mn_g = sg4[2, :N].reshape(N, 1)
    dg_g = sg4[3, :N].reshape(N, 1)

    x1, st1 = _agg_stage(feat, s_g, mx_g, mn_g, dg_g, WO, bO)
    x2, st2 = _ffn_stage(x1, st1, g1, be1, Wf1, bf1, Wf2, bf2)
    return _bn2_stage(x2, st2, g2, be2)


# speedup vs baseline: 1.0034x; 1.0034x over previous
import functools

import jax
import jax.numpy as jnp
from jax import lax
from jax.experimental import pallas as pl
from jax.experimental.pallas import tpu as pltpu
from jax.experimental.pallas import tpu_sc as plsc

N = 10000
D = 128
M5 = 5
NU = 2500
ND = 7500
NU_P = 2560
ND_P = 7680
BD = 640
BN = 2000
EPS = 1e-5
NW = 32
N_P = 10240
PADI = 9999


def _feat_body(h_ref, sl16_ref, wh_ref, wsp_ref, bemb_ref, w1_ref,
               feat_ref, fw1_ref):
    f = (jnp.dot(h_ref[...], wh_ref[...], preferred_element_type=jnp.float32)
         + jnp.dot(sl16_ref[...], wsp_ref[...],
                   preferred_element_type=jnp.float32)
         + bemb_ref[...])
    feat_ref[...] = f
    fw1_ref[...] = jnp.dot(f, w1_ref[...], preferred_element_type=jnp.float32)


def _features(h, sl16, W_emb, b_emb, W1):
    wh = W_emb[:D, :]
    wsp = jnp.zeros((16, D), jnp.float32).at[:3, :].set(W_emb[D:, :])
    grid = N // BN
    return pl.pallas_call(
        _feat_body,
        grid=(grid,),
        in_specs=[
            pl.BlockSpec((BN, D), lambda i: (i, 0)),
            pl.BlockSpec((BN, 16), lambda i: (i, 0)),
            pl.BlockSpec((D, D), lambda i: (0, 0)),
            pl.BlockSpec((16, D), lambda i: (0, 0)),
            pl.BlockSpec((1, D), lambda i: (0, 0)),
            pl.BlockSpec((D, D), lambda i: (0, 0)),
        ],
        out_specs=[
            pl.BlockSpec((BN, D), lambda i: (i, 0)),
            pl.BlockSpec((BN, D), lambda i: (i, 0)),
        ],
        out_shape=[
            jax.ShapeDtypeStruct((N, D), jnp.float32),
            jax.ShapeDtypeStruct((N, D), jnp.float32),
        ],
    )(h, sl16, wh, wsp, b_emb.reshape(1, D), W1)


def _knn_body(sdp_ref, fwd_ref, sut_ref, fwub_ref, b1_ref, w2_ref, b2_ref,
              idx_ref, c_ref):
    sd0 = sdp_ref[:, 0:1]
    sd1 = sdp_ref[:, 1:2]
    sd2c = sdp_ref[:, 2:3]
    su0 = sut_ref[0:1, :]
    su1 = sut_ref[1:2, :]
    su2c = sut_ref[2:3, :]
    sdq = sd0 * sd0 + sd1 * sd1 + sd2c * sd2c
    suq = su0 * su0 + su1 * su1 + su2c * su2c
    cross = jnp.dot(sdp_ref[:, 0:8], sut_ref[...],
                    preferred_element_type=jnp.float32)
    d2 = sdq + suq - 2.0 * cross

    row = pl.program_id(0) * BD + jax.lax.broadcasted_iota(jnp.int32, (BD, 1), 0)
    valid = row < ND
    lane = jax.lax.broadcasted_iota(jnp.int32, (BD, NU_P), 1)

    fwd = fwd_ref[...]
    b1 = b1_ref[...]
    w2 = w2_ref[...]
    b2 = b2_ref[...]

    idx_cols = []
    c_cols = []
    for _ in range(M5):
        idxv = jnp.argmin(d2, axis=1).astype(jnp.int32).reshape(BD, 1)
        sel = lane == idxv
        d2 = jnp.where(sel, 1e30, d2)
        maskf = jnp.where(sel, 1.0, 0.0)
        g = jnp.dot(maskf, fwub_ref[...], preferred_element_type=jnp.float32)
        t = jnp.maximum(fwd - g + b1, 0.0)
        att = jnp.maximum(jnp.dot(t, w2, preferred_element_type=jnp.float32)
                          + b2, 0.0)
        c = jnp.exp(-att)
        idx_cols.append(jnp.where(valid, idxv, NU_P - 1))
        c_cols.append(c)

    zi = jnp.full((BD, 1), NU_P - 1, jnp.int32)
    zf = jnp.zeros((BD, 1), jnp.float32)
    idx_ref[...] = jnp.concatenate(idx_cols + [zi, zi, zi], axis=1)
    c_ref[...] = jnp.concatenate(c_cols + [zf, zf, zf], axis=1)


def _knn_agg(sdp, fwd, sut, fwu, b1, W2, b2):
    grid = ND_P // BD
    return pl.pallas_call(
        _knn_body,
        grid=(grid,),
        in_specs=[
            pl.BlockSpec((BD, 16), lambda i: (i, 0)),
            pl.BlockSpec((BD, D), lambda i: (i, 0)),
            pl.BlockSpec((8, NU_P), lambda i: (0, 0)),
            pl.BlockSpec((NU_P, D), lambda i: (0, 0)),
            pl.BlockSpec((1, D), lambda i: (0, 0)),
            pl.BlockSpec((D, 1), lambda i: (0, 0)),
            pl.BlockSpec((1, 1), lambda i: (0, 0)),
        ],
        out_specs=[
            pl.BlockSpec((BD, 8), lambda i: (i, 0)),
            pl.BlockSpec((BD, 8), lambda i: (i, 0)),
        ],
        out_shape=[
            jax.ShapeDtypeStruct((ND_P, 8), jnp.int32),
            jax.ShapeDtypeStruct((ND_P, 8), jnp.float32),
        ],
    )(sdp, fwd, sut, fwu, b1.reshape(1, D), W2, b2.reshape(1, 1))


def _agg_body(f_ref, s_ref, mx_ref, mn_ref, dg_ref, wo_ref, bo_ref,
              x1_ref, st_ref):
    pid = pl.program_id(0)

    @pl.when(pid == 0)
    def _init():
        st_ref[...] = jnp.zeros_like(st_ref)

    f = f_ref[...]
    s = s_ref[...]
    mx = mx_ref[...]
    mn = mn_ref[...]
    dg = dg_ref[...]
    mean_agg = f * (s / jnp.maximum(dg, 1.0))
    max_agg = f * jnp.where(f >= 0.0, mx, mn)
    cat = jnp.concatenate([mean_agg, max_agg], axis=1)
    agg = jnp.dot(cat, wo_ref[...], preferred_element_type=jnp.float32) \
        + bo_ref[...]
    agg = jnp.where(dg > 0.0, agg, 0.0)
    x1 = f + agg
    x1_ref[...] = x1
    st_ref[0:1, :] += jnp.sum(x1, axis=0, keepdims=True)
    st_ref[1:2, :] += jnp.sum(x1 * x1, axis=0, keepdims=True)


def _agg_stage(feat, s_g, mx_g, mn_g, dg_g, WO, bO):
    grid = N // BN
    return pl.pallas_call(
        _agg_body,
        grid=(grid,),
        in_specs=[
            pl.BlockSpec((BN, D), lambda i: (i, 0)),
            pl.BlockSpec((BN, 1), lambda i: (i, 0)),
            pl.BlockSpec((BN, 1), lambda i: (i, 0)),
            pl.BlockSpec((BN, 1), lambda i: (i, 0)),
            pl.BlockSpec((BN, 1), lambda i: (i, 0)),
            pl.BlockSpec((2 * D, D), lambda i: (0, 0)),
            pl.BlockSpec((1, D), lambda i: (0, 0)),
        ],
        out_specs=[
            pl.BlockSpec((BN, D), lambda i: (i, 0)),
            pl.BlockSpec((8, D), lambda i: (0, 0)),
        ],
        out_shape=[
            jax.ShapeDtypeStruct((N, D), jnp.float32),
            jax.ShapeDtypeStruct((8, D), jnp.float32),
        ],
    )(feat, s_g, mx_g, mn_g, dg_g, WO, bO.reshape(1, D))


def _ffn_body(x1_ref, st_ref, g1_ref, be1_ref, wf1_ref, bf1_ref, wf2_ref,
              bf2_ref, x2_ref, st2_ref):
    pid = pl.program_id(0)

    @pl.when(pid == 0)
    def _init():
        st2_ref[...] = jnp.zeros_like(st2_ref)

    mu = st_ref[0:1, :] / N
    var = st_ref[1:2, :] / N - mu * mu
    hh = (x1_ref[...] - mu) / jnp.sqrt(var + EPS) * g1_ref[...] + be1_ref[...]
    t = jnp.maximum(jnp.dot(hh, wf1_ref[...],
                            preferred_element_type=jnp.float32)
                    + bf1_ref[...], 0.0)
    y = jnp.dot(t, wf2_ref[...], preferred_element_type=jnp.float32) \
        + bf2_ref[...]
    x2 = hh + y
    x2_ref[...] = x2
    st2_ref[0:1, :] += jnp.sum(x2, axis=0, keepdims=True)
    st2_ref[1:2, :] += jnp.sum(x2 * x2, axis=0, keepdims=True)


def _ffn_stage(x1, st1, g1, be1, Wf1, bf1, Wf2, bf2):
    grid = N // BN
    return pl.pallas_call(
        _ffn_body,
        grid=(grid,),
        in_specs=[
            pl.BlockSpec((BN, D), lambda i: (i, 0)),
            pl.BlockSpec((8, D), lambda i: (0, 0)),
            pl.BlockSpec((1, D), lambda i: (0, 0)),
            pl.BlockSpec((1, D), lambda i: (0, 0)),
            pl.BlockSpec((D, 2 * D), lambda i: (0, 0)),
            pl.BlockSpec((1, 2 * D), lambda i: (0, 0)),
            pl.BlockSpec((2 * D, D), lambda i: (0, 0)),
            pl.BlockSpec((1, D), lambda i: (0, 0)),
        ],
        out_specs=[
            pl.BlockSpec((BN, D), lambda i: (i, 0)),
            pl.BlockSpec((8, D), lambda i: (0, 0)),
        ],
        out_shape=[
            jax.ShapeDtypeStruct((N, D), jnp.float32),
            jax.ShapeDtypeStruct((8, D), jnp.float32),
        ],
    )(x1, st1, g1.reshape(1, D), be1.reshape(1, D), Wf1,
      bf1.reshape(1, 2 * D), Wf2, bf2.reshape(1, D))


def _bn2_body(x2_ref, st_ref, g2_ref, be2_ref, out_ref):
    mu = st_ref[0:1, :] / N
    var = st_ref[1:2, :] / N - mu * mu
    out_ref[...] = (x2_ref[...] - mu) / jnp.sqrt(var + EPS) * g2_ref[...] \
        + be2_ref[...]


def _bn2_stage(x2, st2, g2, be2):
    grid = N // BN
    return pl.pallas_call(
        _bn2_body,
        grid=(grid,),
        in_specs=[
            pl.BlockSpec((BN, D), lambda i: (i, 0)),
            pl.BlockSpec((8, D), lambda i: (0, 0)),
            pl.BlockSpec((1, D), lambda i: (0, 0)),
            pl.BlockSpec((1, D), lambda i: (0, 0)),
        ],
        out_specs=pl.BlockSpec((BN, D), lambda i: (i, 0)),
        out_shape=jax.ShapeDtypeStruct((N, D), jnp.float32),
    )(x2, st2, g2.reshape(1, D), be2.reshape(1, D))


def _thr_compute(bits_ref, out_ref):
    bits = bits_ref[...]
    ridx = jax.lax.broadcasted_iota(jnp.int32, (N_P, 1), 0)
    j128 = jax.lax.broadcasted_iota(jnp.int32, (1, 128), 1)

    def count_ge(tvec):
        ge = jnp.where(bits >= tvec, 1.0, 0.0)
        return jnp.sum(ge, axis=0, keepdims=True)

    lo = jnp.int32(0)
    for step in (1 << 23, 1 << 16, 1 << 9, 1 << 2, 1):
        tvec = lo + j128 * step
        cnt = count_ge(tvec)
        jstar = jnp.max(jnp.where(cnt >= NU, j128, 0))
        lo = lo + jstar * step
    t = lo
    n_gt = jnp.sum(jnp.where(bits >= t + 1, 1.0, 0.0)).astype(jnp.int32)
    need = NU - n_gt
    tie = bits == t

    lo2 = jnp.int32(0)
    for step in (128, 1):
        xvec = lo2 + j128 * step
        cnt = jnp.sum(jnp.where(tie & (ridx >= xvec), 1.0, 0.0),
                      axis=0, keepdims=True)
        jstar = jnp.max(jnp.where(cnt >= need, j128, 0))
        lo2 = lo2 + jstar * step

    out_ref[0:1, :] = jnp.full((1, 128), t, jnp.int32)
    out_ref[1:2, :] = jnp.full((1, 128), lo2, jnp.int32)


def _threshold(bits_col):
    return pl.pallas_call(
        _thr_compute,
        grid=(1,),
        in_specs=[pl.BlockSpec((N_P, 1), lambda i: (0, 0))],
        out_specs=pl.BlockSpec((8, 128), lambda i: (0, 0)),
        out_shape=jax.ShapeDtypeStruct((8, 128), jnp.int32),
    )(bits_col)


def _sc_wid():
    return lax.axis_index("s") * 2 + lax.axis_index("c")


def _compact(bits, thr):
    mesh = plsc.VectorSubcoreMesh(core_axis_name="c", subcore_axis_name="s")

    @functools.partial(
        pl.kernel, mesh=mesh,
        out_type=[
            jax.ShapeDtypeStruct((1, NU_P), jnp.int32),
            jax.ShapeDtypeStruct((1, ND_P), jnp.int32),
        ],
        compiler_params=pltpu.CompilerParams(needs_layout_passes=False),
        scratch_types=[
            pltpu.VMEM((N,), jnp.int32),
            pltpu.VMEM((16,), jnp.int32),
            pltpu.VMEM((16,), jnp.int32),
            pltpu.VMEM((1, NU_P), jnp.int32),
            pltpu.VMEM((1, ND_P), jnp.int32),
        ],
    )
    def k(bits_hbm, thr_hbm, up_hbm, dn_hbm, bits_v, t_v, x_v, up_v, dn_v):
        w = _sc_wid()

        @pl.when(w == 0)
        def _():
            pltpu.sync_copy(bits_hbm, bits_v)
            pltpu.sync_copy(thr_hbm.at[pl.ds(0, 16)], t_v)
            pltpu.sync_copy(thr_hbm.at[pl.ds(128, 16)], x_v)
            t = t_v[...]
            x = x_v[...]
            lanes = lax.iota(jnp.int32, 16)
            zrow = jnp.zeros((16,), jnp.int32)
            padv = jnp.full((16,), PADI, jnp.int32)
            ones = jnp.full((16,), 1, jnp.int32)

            def fill_up(j, _):
                plsc.store_scatter(up_v, [zrow, lanes + j * 16], padv)
                return 0

            def fill_dn(j, _):
                plsc.store_scatter(dn_v, [zrow, lanes + j * 16], padv)
                return 0

            lax.fori_loop(0, NU_P // 16, fill_up, 0)
            lax.fori_loop(0, ND_P // 16, fill_dn, 0)

            def body(i, offs):
                uo, do = offs
                b16 = bits_v[pl.ds(i * 16, 16)]
                idx16 = lanes + i * 16
                m_up = (b16 > t) | ((b16 == t) & (idx16 >= x))
                m_dn = ~m_up
                cu = plsc.cumsum(m_up.astype(jnp.int32))
                cd = plsc.cumsum(m_dn.astype(jnp.int32))
                plsc.store_scatter(up_v, [zrow, uo + cu - 1], idx16,
                                   mask=m_up)
                plsc.store_scatter(dn_v, [zrow, do + cd - 1], idx16,
                                   mask=m_dn)
                nu16 = jnp.sum(jnp.where(m_up, ones, 0))
                return (uo + nu16, do + (16 - nu16))

            lax.fori_loop(0, N // 16, body,
                          (jnp.int32(0), jnp.int32(0)))
            pltpu.sync_copy(up_v, up_hbm)
            pltpu.sync_copy(dn_v, dn_hbm)

    return k(bits, thr)


def _sc_gather3(fw1, up_idx, dn_idx):
    bu = NU_P // NW
    bd = ND_P // NW
    mesh = plsc.VectorSubcoreMesh(core_axis_name="c", subcore_axis_name="s")

    @functools.partial(
        pl.kernel, mesh=mesh,
        out_type=[
            jax.ShapeDtypeStruct((NU_P, D), jnp.float32),
            jax.ShapeDtypeStruct((ND_P, D), jnp.float32),
        ],
        scratch_types=[
            pltpu.VMEM((bu,), jnp.int32),
            pltpu.VMEM((bd,), jnp.int32),
            pltpu.VMEM((bu, D), jnp.float32),
            pltpu.VMEM((bd, D), jnp.float32),
            pltpu.SemaphoreType.DMA,
        ],
    )
    def k(fw1_hbm, up_hbm, dn_hbm, fwu_hbm, fwd_hbm,
          iu_v, id_v, ru_v, rd_v, sem):
        w = _sc_wid()
        pltpu.sync_copy(up_hbm.at[pl.ds(w * bu, bu)], iu_v)
        pltpu.sync_copy(dn_hbm.at[pl.ds(w * bd, bd)], id_v)
        cu = pltpu.async_copy(fw1_hbm.at[iu_v], ru_v, sem)
        cd = pltpu.async_copy(fw1_hbm.at[id_v], rd_v, sem)
        cu.wait()
        cd.wait()
        pltpu.sync_copy(ru_v, fwu_hbm.at[pl.ds(w * bu, bu)])
        pltpu.sync_copy(rd_v, fwd_hbm.at[pl.ds(w * bd, bd)])

    return k(fw1, up_idx, dn_idx)


def _take16(x, i):
    dn = lax.GatherDimensionNumbers(offset_dims=(), collapsed_slice_dims=(0,),
                                    start_index_map=(0,))
    return lax.gather(x, i[:, None], dn, slice_sizes=(1,),
                      mode=lax.GatherScatterMode.PROMISE_IN_BOUNDS)


def _sc_segagg(idx5, c5):
    ew = ND_P * 8 // NW
    nit = ew // 16
    mesh = plsc.VectorSubcoreMesh(core_axis_name="c", subcore_axis_name="s")

    @functools.partial(
        pl.kernel, mesh=mesh,
        out_type=jax.ShapeDtypeStruct((NW * 4, NU_P), jnp.float32),
        compiler_params=pltpu.CompilerParams(needs_layout_passes=False),
        scratch_types=[
            pltpu.VMEM((ew,), jnp.int32),
            pltpu.VMEM((ew,), jnp.float32),
            pltpu.VMEM((4, NU_P), jnp.float32),
        ],
    )
    def k(idx_hbm, c_hbm, out_hbm, idx_v, c_v, part_v):
        w = _sc_wid()
        pltpu.sync_copy(idx_hbm.at[pl.ds(w * ew, ew)], idx_v)
        pltpu.sync_copy(c_hbm.at[pl.ds(w * ew, ew)], c_v)
        lanes = lax.iota(jnp.int32, 16)
        zrow = jnp.zeros((16,), jnp.int32)
        onerow = jnp.full((16,), 1, jnp.int32)
        tworow = jnp.full((16,), 2, jnp.int32)
        threerow = jnp.full((16,), 3, jnp.int32)
        onesf = jnp.full((16,), 1.0, jnp.float32)
        zz = jnp.zeros((16,), jnp.float32)
        tt = jnp.full((16,), 2.0, jnp.float32)
        lmask = (lanes % 8) < M5

        for j in range(NU_P // 16):
            part_v[0, pl.ds(j * 16, 16)] = zz
            part_v[1, pl.ds(j * 16, 16)] = zz
            part_v[2, pl.ds(j * 16, 16)] = tt
            part_v[3, pl.ds(j * 16, 16)] = zz

        def shift_ok(sk, d):
            idn = jnp.maximum(lanes - d, 0)
            km = (_take16(sk, idn) == sk) \
                & (lanes >= d)
            return idn, km

        def body(i, _):
            ii = idx_v[pl.ds(i * 16, 16)]
            cc = c_v[pl.ds(i * 16, 16)]
            plsc.addupdate_scatter(part_v, [zrow, ii], cc, mask=lmask)
            plsc.addupdate_scatter(part_v, [threerow, ii], onesf, mask=lmask)
            key = jnp.where(lmask, ii, NU_P - 1)
            sk, smx = plsc.sort_key_val(key, jnp.where(lmask, cc, 0.0))
            _, smn = plsc.sort_key_val(key, jnp.where(lmask, cc, 2.0))
            for d in (1, 2, 4, 8):
                idn, km = shift_ok(sk, d)
                smx = jnp.maximum(
                    smx, jnp.where(km, _take16(smx, idn),
                                   0.0))
                smn = jnp.minimum(
                    smn, jnp.where(km, _take16(smn, idn),
                                   2.0))
            nxt = jnp.minimum(lanes + 1, 15)
            islast = (lanes == 15) | (
                _take16(sk, nxt) != sk)
            curx = plsc.load_gather(part_v, [onerow, sk], mask=islast)
            plsc.store_scatter(part_v, [onerow, sk],
                               jnp.maximum(curx, smx), mask=islast)
            curn = plsc.load_gather(part_v, [tworow, sk], mask=islast)
            plsc.store_scatter(part_v, [tworow, sk],
                               jnp.minimum(curn, smn), mask=islast)
            return 0

        lax.fori_loop(0, nit, body, 0)
        pltpu.sync_copy(part_v, out_hbm.at[pl.ds(w * 4, 4)])

    return k(idx5, c5)


def _comb_body(p_ref, acc_ref):
    s = jnp.zeros((1, NU_P), jnp.float32)
    mx = jnp.zeros((1, NU_P), jnp.float32)
    mn = jnp.full((1, NU_P), 2.0, jnp.float32)
    dg = jnp.zeros((1, NU_P), jnp.float32)
    for w in range(NW):
        s = s + p_ref[4 * w:4 * w + 1, :]
        mx = jnp.maximum(mx, p_ref[4 * w + 1:4 * w + 2, :])
        mn = jnp.minimum(mn, p_ref[4 * w + 2:4 * w + 3, :])
        dg = dg + p_ref[4 * w + 3:4 * w + 4, :]
    acc_ref[0:1, :] = s
    acc_ref[1:2, :] = mx
    acc_ref[2:3, :] = mn
    acc_ref[3:4, :] = dg


def _combine(parts):
    return pl.pallas_call(
        _comb_body,
        grid=(1,),
        in_specs=[pl.BlockSpec((NW * 4, NU_P), lambda i: (0, 0))],
        out_specs=pl.BlockSpec((4, NU_P), lambda i: (0, 0)),
        out_shape=jax.ShapeDtypeStruct((4, NU_P), jnp.float32),
    )(parts)


def _sc_scatter4(acc, up_idx):
    bcols = N_P // NW
    mesh = plsc.VectorSubcoreMesh(core_axis_name="c", subcore_axis_name="s")

    @functools.partial(
        pl.kernel, mesh=mesh,
        out_type=jax.ShapeDtypeStruct((4, NW, bcols), jnp.float32),
        compiler_params=pltpu.CompilerParams(needs_layout_passes=False),
        scratch_types=[
            pltpu.VMEM((NU_P,), jnp.int32),
            pltpu.VMEM((4, NU_P), jnp.float32),
            pltpu.VMEM((4, bcols), jnp.float32),
        ],
    )
    def k(acc_hbm, up_hbm, out_hbm, idx_v, acc_v, buf_v):
        w = _sc_wid()
        lo = w * bcols
        pltpu.sync_copy(up_hbm, idx_v)
        pltpu.sync_copy(acc_hbm, acc_v)
        zz = jnp.zeros((16,), jnp.float32)
        lanes = lax.iota(jnp.int32, 16)
        for kk in range(4):
            for j in range(bcols // 16):
                buf_v[kk, pl.ds(j * 16, 16)] = zz

        def body(i, _):
            idx = idx_v[pl.ds(i * 16, 16)]
            pos = lanes + i * 16
            rel = idx - lo
            msk = (rel >= 0) & (rel < bcols) & (pos < NU)
            for kk in range(4):
                val = acc_v[kk, pl.ds(i * 16, 16)]
                plsc.store_scatter(buf_v,
                                   [jnp.full((16,), kk, jnp.int32), rel],
                                   val, mask=msk)
            return 0

        lax.fori_loop(0, NU_P // 16, body, 0)
        pltpu.sync_copy(buf_v, out_hbm.at[:, w])

    return k(acc, up_idx)


def kernel(h, s_l, scores, W_emb, b_emb, W1, b1, W2, b2, WO, bO, g1, be1,
           Wf1, bf1, Wf2, bf2, g2, be2):
    bits = jax.lax.bitcast_convert_type(scores, jnp.int32)
    bits_col = jnp.pad(bits, (0, N_P - N),
                       constant_values=-2**30).reshape(N_P, 1)
    sl16 = jnp.zeros((N, 16), jnp.float32).at[:, :3].set(s_l)

    thr = _threshold(bits_col).reshape(1024)
    nodes_up_p, nodes_down_p = _compact(bits, thr)

    feat, fw1 = _features(h, sl16, W_emb, b_emb, W1)
    nodes_up_p = nodes_up_p.reshape(NU_P)
    nodes_down_p = nodes_down_p.reshape(ND_P)
    nodes_up = nodes_up_p[:NU]

    sut = jnp.zeros((8, NU_P), jnp.float32)
    sut = sut.at[:3, :].set(1e6)
    sut = sut.at[:3, :NU].set(jnp.take(s_l, nodes_up, axis=0).T)
    fwu, fwd = _sc_gather3(fw1, nodes_up_p, nodes_down_p)
    sdp = jnp.take(sl16, nodes_down_p, axis=0)

    idx5, c5 = _knn_agg(sdp, fwd, sut, fwu, b1, W2, b2)
    parts = _sc_segagg(idx5.reshape(ND_P * 8), c5.reshape(ND_P * 8))
    acc = _combine(parts)

    sg4 = _sc_scatter4(acc, nodes_up_p).reshape(4, N_P)
    s_g = sg4[0, :N].reshape(N, 1)
    mx_g = sg4[1, :N].reshape(N, 1)
    mn_g = sg4[2, :N].reshape(N, 1)
    dg_g = sg4[3, :N].reshape(N, 1)

    x1, st1 = _agg_stage(feat, s_g, mx_g, mn_g, dg_g, WO, bO)
    x2, st2 = _ffn_stage(x1, st1, g1, be1, Wf1, bf1, Wf2, bf2)
    return _bn2_stage(x2, st2, g2, be2)
